# Initial kernel scaffold; baseline (speedup 1.0000x reference)
import jax
import jax.numpy as jnp
from jax.experimental import pallas as pl


def kernel(u_features, v_features, edge_index, edge_type, edge_val, labels, user_indices, item_indices, u_features_side, v_features_side, W, Wf_u, bf_u, Wf_v, bf_v, Wd_u, Wd_v, P_basis, a_coef):
    raise NotImplementedError("write your pallas kernel here")


# speedup vs baseline: 2.0827x; 2.0827x over previous
import functools

import jax
import jax.numpy as jnp
from jax import lax
from jax.experimental import pallas as pl
from jax.experimental.pallas import tpu as pltpu
from jax.experimental.pallas import tpu_sc as plsc

U = 10000
V = 10000
IN = 256
S = 5
H0 = 500
H1 = 75
C = 100
SIDE = 128
FH = 64
NB = 2
NC = 5
E = 160000
P = 100000

CP = 128
H0P = S * CP
H1P = 128

BR = 1000
f32 = jnp.float32


def _pre_body(x_ref, side_ref, wpad_ref, wf_ref, bf_ref, tmp_ref, feat_ref):
    tmp_ref[...] = jnp.dot(x_ref[...], wpad_ref[...],
                           preferred_element_type=f32)
    f = jnp.dot(side_ref[...], wf_ref[...], preferred_element_type=f32)
    feat_ref[...] = jnp.maximum(f + bf_ref[...], 0.0)


def _pre(x, side, wpad, wf, bf2d):
    n = x.shape[0]
    return pl.pallas_call(
        _pre_body,
        grid=(n // BR,),
        in_specs=[
            pl.BlockSpec((BR, IN), lambda i: (i, 0)),
            pl.BlockSpec((BR, SIDE), lambda i: (i, 0)),
            pl.BlockSpec((IN, H0P), lambda i: (0, 0)),
            pl.BlockSpec((SIDE, FH), lambda i: (0, 0)),
            pl.BlockSpec((1, FH), lambda i: (0, 0)),
        ],
        out_specs=[
            pl.BlockSpec((BR, H0P), lambda i: (i, 0)),
            pl.BlockSpec((BR, FH), lambda i: (i, 0)),
        ],
        out_shape=[
            jax.ShapeDtypeStruct((n, H0P), f32),
            jax.ShapeDtypeStruct((n, FH), f32),
        ],
    )(x, side, wpad, wf, bf2d)


def _post_u_body(z_ref, feat_ref, wd_ref, wdf_ref, p0_ref, p1_ref, out_ref):
    g = jnp.maximum(z_ref[...], 0.0)
    emb = (jnp.dot(g, wd_ref[...], preferred_element_type=f32)
           + jnp.dot(feat_ref[...], wdf_ref[...], preferred_element_type=f32))
    out_ref[:, :H1P] = jnp.dot(emb, p0_ref[...], preferred_element_type=f32)
    out_ref[:, H1P:] = jnp.dot(emb, p1_ref[...], preferred_element_type=f32)


def _post_u(z, feat, wd, wdf, p0, p1):
    return pl.pallas_call(
        _post_u_body,
        grid=(U // BR,),
        in_specs=[
            pl.BlockSpec((BR, H0P), lambda i: (i, 0)),
            pl.BlockSpec((BR, FH), lambda i: (i, 0)),
            pl.BlockSpec((H0P, H1P), lambda i: (0, 0)),
            pl.BlockSpec((FH, H1P), lambda i: (0, 0)),
            pl.BlockSpec((H1P, H1P), lambda i: (0, 0)),
            pl.BlockSpec((H1P, H1P), lambda i: (0, 0)),
        ],
        out_specs=[pl.BlockSpec((BR, 2 * H1P), lambda i: (i, 0))],
        out_shape=[jax.ShapeDtypeStruct((U, 2 * H1P), f32)],
    )(z, feat, wd, wdf, p0, p1)[0]


def _post_v_body(z_ref, feat_ref, wd_ref, wdf_ref, out_ref):
    g = jnp.maximum(z_ref[...], 0.0)
    out_ref[...] = (jnp.dot(g, wd_ref[...], preferred_element_type=f32)
                    + jnp.dot(feat_ref[...], wdf_ref[...],
                              preferred_element_type=f32))


def _post_v(z, feat, wd, wdf):
    return pl.pallas_call(
        _post_v_body,
        grid=(V // BR,),
        in_specs=[
            pl.BlockSpec((BR, H0P), lambda i: (i, 0)),
            pl.BlockSpec((BR, FH), lambda i: (i, 0)),
            pl.BlockSpec((H0P, H1P), lambda i: (0, 0)),
            pl.BlockSpec((FH, H1P), lambda i: (0, 0)),
        ],
        out_specs=[pl.BlockSpec((BR, H1P), lambda i: (i, 0))],
        out_shape=[jax.ShapeDtypeStruct((V, H1P), f32)],
    )(z, feat, wd, wdf)[0]


DBR = 1000


def _dec_body(uh_ref, vb_ref, lab_ref, ac_ref, out_ref, loss_ref):
    i = pl.program_id(0)
    uh = uh_ref[...]
    vb = vb_ref[...]
    d0 = jnp.sum(uh[:, :H1P] * vb, axis=1, keepdims=True)
    d1 = jnp.sum(uh[:, H1P:] * vb, axis=1, keepdims=True)
    logits = d0 * ac_ref[0:1, :] + d1 * ac_ref[1:2, :]
    lane = lax.broadcasted_iota(jnp.int32, logits.shape, 1)
    valid = lane < NC
    masked = jnp.where(valid, logits, -1e30)
    m = jnp.max(masked, axis=1, keepdims=True)
    se = jnp.sum(jnp.where(valid, jnp.exp(masked - m), 0.0),
                 axis=1, keepdims=True)
    lse = m + jnp.log(se)
    picked = jnp.sum(jnp.where(lane == lab_ref[...], logits, 0.0),
                     axis=1, keepdims=True)
    out_ref[...] = logits[:, :8]

    @pl.when(i == 0)
    def _():
        loss_ref[...] = jnp.zeros_like(loss_ref)

    loss_ref[...] += jnp.sum(lse - picked)[None, None] / P


def _decode(uh, vb, lab2d, ac_pad):
    return pl.pallas_call(
        _dec_body,
        grid=(P // DBR,),
        in_specs=[
            pl.BlockSpec((DBR, 2 * H1P), lambda i: (i, 0)),
            pl.BlockSpec((DBR, H1P), lambda i: (i, 0)),
            pl.BlockSpec((DBR, 1), lambda i: (i, 0)),
            pl.BlockSpec((8, H1P), lambda i: (0, 0)),
        ],
        out_specs=[
            pl.BlockSpec((DBR, 8), lambda i: (i, 0)),
            pl.BlockSpec((1, 1), lambda i: (0, 0)),
        ],
        out_shape=[
            jax.ShapeDtypeStruct((P, 8), f32),
            jax.ShapeDtypeStruct((1, 1), f32),
        ],
    )(uh, vb, lab2d, ac_pad)


def _edge_agg(tmp_u, tmp_v, eu, ev, et, val):
    du = eu * S + et
    dv = ev * S + et
    msg_u = val[:, None] * tmp_v[dv]
    z_u = jnp.zeros((U * S, CP), f32).at[du].add(msg_u)
    msg_v = val[:, None] * tmp_u[du]
    z_v = jnp.zeros((V * S, CP), f32).at[dv].add(msg_v)
    return z_u, z_v


def _pair_gather(a_cat, emb_v, ui, ii):
    return a_cat[ui], emb_v[ii]


def kernel(u_features, v_features, edge_index, edge_type, edge_val, labels,
           user_indices, item_indices, u_features_side, v_features_side,
           W, Wf_u, bf_u, Wf_v, bf_v, Wd_u, Wd_v, P_basis, a_coef):
    wpad = jnp.pad(W.reshape(IN, S, C), ((0, 0), (0, 0), (0, CP - C)))
    wpad = wpad.reshape(IN, H0P)
    def pad_wd(Wd):
        wg = jnp.pad(Wd[:H0].reshape(S, C, H1), ((0, 0), (0, CP - C),
                                                 (0, H1P - H1)))
        wf = jnp.pad(Wd[H0:], ((0, 0), (0, H1P - H1)))
        return wg.reshape(H0P, H1P), wf
    wd_u, wdf_u = pad_wd(Wd_u)
    wd_v, wdf_v = pad_wd(Wd_v)
    p0 = jnp.pad(P_basis[0], ((0, H1P - H1), (0, H1P - H1)))
    p1 = jnp.pad(P_basis[1], ((0, H1P - H1), (0, H1P - H1)))
    ac_pad = jnp.zeros((8, H1P), f32).at[:NB, :NC].set(a_coef)
    bf_u2 = bf_u.reshape(1, FH)
    bf_v2 = bf_v.reshape(1, FH)

    eu = edge_index[0].astype(jnp.int32)
    ev = edge_index[1].astype(jnp.int32)
    et = edge_type.astype(jnp.int32)
    ui = user_indices.astype(jnp.int32)
    ii = item_indices.astype(jnp.int32)
    lab2d = labels.astype(jnp.int32).reshape(P, 1)

    tmp_u, feat_u = _pre(u_features, u_features_side, wpad, Wf_u, bf_u2)
    tmp_v, feat_v = _pre(v_features, v_features_side, wpad, Wf_v, bf_v2)

    z_u, z_v = _edge_agg(tmp_u.reshape(U * S, CP), tmp_v.reshape(V * S, CP),
                         eu, ev, et, edge_val)

    a_cat = _post_u(z_u.reshape(U, H0P), feat_u, wd_u, wdf_u, p0, p1)
    emb_v = _post_v(z_v.reshape(V, H0P), feat_v, wd_v, wdf_v)

    uh, vbm = _pair_gather(a_cat, emb_v, ui, ii)

    out8, loss11 = _decode(uh, vbm, lab2d, ac_pad)
    return out8[:, :NC], loss11.reshape(())
